# baseline (speedup 1.0000x reference)
import functools

import jax
import jax.numpy as jnp
from jax import lax
from jax.experimental import pallas as pl
from jax.experimental.pallas import tpu as pltpu
from jax.experimental.pallas import tpu_sc as plsc

N = 10000
E = 320000
D_IN = 128
D_H = 64
D_OUT = 32

NC = 2
NS = 16
CH = 128
K = 80
TOT = NC * NS * K * CH
N_PAD = 10240
RPT = N_PAD // NS
NBUF = 8

_mesh = plsc.VectorSubcoreMesh(core_axis_name="c", subcore_axis_name="s")


def _sc_degree(dsti, zeros1):
    @functools.partial(
        pl.kernel,
        mesh=_mesh,
        compiler_params=pltpu.CompilerParams(use_tc_tiling_on_sc=False),
        out_type=jax.ShapeDtypeStruct((NC, N_PAD), jnp.float32),
        scratch_types=[
            pltpu.VMEM((K, CH), jnp.int32),
            pltpu.VMEM((CH,), jnp.float32),
            pltpu.VMEM_SHARED((N_PAD,), jnp.float32),
        ],
    )
    def k(dst_h, z_h, out, dst_v, ones_v, dacc):
        cid = lax.axis_index("c")
        sid = lax.axis_index("s")
        pltpu.sync_copy(dst_h.at[cid, sid], dst_v)
        for t in range(CH // 16):
            ones_v[pl.ds(16 * t, 16)] = jnp.full((16,), 1.0, jnp.float32)
        r0 = sid * RPT
        pltpu.sync_copy(z_h.at[pl.ds(r0, RPT)], dacc.at[pl.ds(r0, RPT)])
        plsc.subcore_barrier()

        def body(j, carry):
            pltpu.sync_copy(ones_v, dacc.at[dst_v.at[j]], add=True)
            return carry

        lax.fori_loop(0, K, body, 0)
        plsc.subcore_barrier()
        pltpu.sync_copy(dacc.at[pl.ds(r0, RPT)], out.at[cid, pl.ds(r0, RPT)])

    return k(dsti, zeros1)


def _sc_edge_pass(y, srci, dsti, zeros2, d):
    @functools.partial(
        pl.kernel,
        mesh=_mesh,
        compiler_params=pltpu.CompilerParams(use_tc_tiling_on_sc=False),
        out_type=jax.ShapeDtypeStruct((NC, N_PAD, d), jnp.float32),
        scratch_types=[
            pltpu.VMEM((K, CH), jnp.int32),
            pltpu.VMEM((K, CH), jnp.int32),
            [pltpu.VMEM((CH, d), jnp.float32) for _ in range(NBUF)],
            pltpu.VMEM_SHARED((N_PAD, d), jnp.float32),
            [pltpu.SemaphoreType.DMA for _ in range(NBUF)],
        ],
    )
    def k(y_h, src_h, dst_h, z_h, out, src_v, dst_v, rows, acc, sems):
        cid = lax.axis_index("c")
        sid = lax.axis_index("s")
        pltpu.sync_copy(src_h.at[cid, sid], src_v)
        pltpu.sync_copy(dst_h.at[cid, sid], dst_v)
        r0 = sid * RPT
        pltpu.sync_copy(z_h.at[pl.ds(r0, RPT)], acc.at[pl.ds(r0, RPT)])
        plsc.subcore_barrier()

        for b in range(NBUF):
            pltpu.make_async_copy(y_h.at[src_v.at[b]], rows[b], sems[b]).start()

        def body(g, carry):
            j = NBUF * g
            for b in range(NBUF):
                pltpu.make_async_copy(
                    y_h.at[src_v.at[j + b]], rows[b], sems[b]).wait()
                pltpu.sync_copy(rows[b], acc.at[dst_v.at[j + b]], add=True)
                pltpu.make_async_copy(
                    y_h.at[src_v.at[j + b + NBUF]], rows[b], sems[b]).start()
            return carry

        lax.fori_loop(0, K // NBUF - 1, body, 0)
        for b in range(NBUF):
            j = K - NBUF + b
            pltpu.make_async_copy(y_h.at[src_v.at[j]], rows[b], sems[b]).wait()
            pltpu.sync_copy(rows[b], acc.at[dst_v.at[j]], add=True)

        plsc.subcore_barrier()
        pltpu.sync_copy(acc.at[pl.ds(r0, RPT)], out.at[cid, pl.ds(r0, RPT)])

    return k(y, srci, dsti, zeros2)


def _tc_mm(x_pad, W1):
    def body(x_ref, w_ref, o_ref):
        o_ref[...] = jnp.dot(x_ref[...], w_ref[...],
                             preferred_element_type=jnp.float32)

    return pl.pallas_call(
        body,
        out_shape=jax.ShapeDtypeStruct((N_PAD, D_H), jnp.float32),
    )(x_pad, W1)


def _tc_scale(deg2, xw):
    def body(deg_ref, xw_ref, d_ref, y_ref):
        deg = deg_ref[0] + deg_ref[1] + 1.0
        dcol = lax.rsqrt(deg)
        d_ref[...] = dcol
        y_ref[...] = xw_ref[...] * dcol

    return pl.pallas_call(
        body,
        out_shape=[
            jax.ShapeDtypeStruct((N_PAD, 1), jnp.float32),
            jax.ShapeDtypeStruct((N_PAD, D_H), jnp.float32),
        ],
    )(deg2, xw)


def _tc_mid(acc1, y1, d, b1, W2):
    def body(acc_ref, y_ref, d_ref, b_ref, w_ref, y2_ref):
        s = acc_ref[0] + acc_ref[1]
        dcol = d_ref[...]
        h = jnp.maximum((s + y_ref[...]) * dcol + b_ref[...], 0.0)
        y2_ref[...] = jnp.dot(h, w_ref[...],
                              preferred_element_type=jnp.float32) * dcol

    return pl.pallas_call(
        body,
        out_shape=jax.ShapeDtypeStruct((N_PAD, D_OUT), jnp.float32),
    )(acc1, y1, d, b1.reshape(1, D_H), W2)


def _tc_final(acc2, y2, d, b2):
    def body(acc_ref, y_ref, d_ref, b_ref, o_ref):
        s = acc_ref[0] + acc_ref[1]
        o_ref[...] = jnp.maximum(
            (s + y_ref[...]) * d_ref[...] + b_ref[...], 0.0)

    return pl.pallas_call(
        body,
        out_shape=jax.ShapeDtypeStruct((N_PAD, D_OUT), jnp.float32),
    )(acc2, y2, d, b2.reshape(1, D_OUT))


def kernel(x, edge_index, W1, b1, W2, b2):
    src = edge_index[0]
    dst = edge_index[1]
    pad = TOT - E
    src_p = jnp.concatenate([src, jnp.zeros((pad,), jnp.int32)])
    dst_p = jnp.concatenate([dst, jnp.full((pad,), N, jnp.int32)])
    srci = src_p.reshape(NC, NS, K, CH)
    dsti = dst_p.reshape(NC, NS, K, CH)

    x_pad = jnp.pad(x, ((0, N_PAD - N), (0, 0)))
    z1 = jnp.zeros((N_PAD,), jnp.float32)
    zH = jnp.zeros((N_PAD, D_H), jnp.float32)
    zO = jnp.zeros((N_PAD, D_OUT), jnp.float32)

    deg2 = _sc_degree(dsti, z1)
    xw = _tc_mm(x_pad, W1)
    d, y1 = _tc_scale(deg2.reshape(NC, N_PAD, 1), xw)
    acc1 = _sc_edge_pass(y1, srci, dsti, zH, D_H)
    y2 = _tc_mid(acc1, y1, d, b1, W2)
    acc2 = _sc_edge_pass(y2, srci, dsti, zO, D_OUT)
    out = _tc_final(acc2, y2, d, b2)
    return out[:N]


# speedup vs baseline: 2.0972x; 2.0972x over previous
import functools

import jax
import jax.numpy as jnp
from jax import lax
from jax.experimental import pallas as pl
from jax.experimental.pallas import tpu as pltpu
from jax.experimental.pallas import tpu_sc as plsc

N = 10000
E = 320000
D_IN = 128
D_H = 64
D_OUT = 32

NC = 2
NS = 16
CH = 128
KD = 80
KE = 160
N_PAD = 10240
RPT = N_PAD // NS
NBUF = 8

HH = D_H // NC
HO = D_OUT // NC

_mesh = plsc.VectorSubcoreMesh(core_axis_name="c", subcore_axis_name="s")


def _sc_degree(dsti, zeros1):
    @functools.partial(
        pl.kernel,
        mesh=_mesh,
        compiler_params=pltpu.CompilerParams(use_tc_tiling_on_sc=False),
        out_type=jax.ShapeDtypeStruct((NC, N_PAD), jnp.float32),
        scratch_types=[
            pltpu.VMEM((KD, CH), jnp.int32),
            pltpu.VMEM((CH,), jnp.float32),
            pltpu.VMEM_SHARED((N_PAD,), jnp.float32),
        ],
    )
    def k(dst_h, z_h, out, dst_v, ones_v, dacc):
        cid = lax.axis_index("c")
        sid = lax.axis_index("s")
        pltpu.sync_copy(dst_h.at[cid, sid], dst_v)
        for t in range(CH // 16):
            ones_v[pl.ds(16 * t, 16)] = jnp.full((16,), 1.0, jnp.float32)
        r0 = sid * RPT
        pltpu.sync_copy(z_h.at[pl.ds(r0, RPT)], dacc.at[pl.ds(r0, RPT)])
        plsc.subcore_barrier()

        def body(j, carry):
            pltpu.sync_copy(ones_v, dacc.at[dst_v.at[j]], add=True)
            return carry

        lax.fori_loop(0, KD, body, 0)
        plsc.subcore_barrier()
        pltpu.sync_copy(dacc.at[pl.ds(r0, RPT)], out.at[cid, pl.ds(r0, RPT)])

    return k(dsti, zeros1)


def _sc_edge_pass(y2h, srci, dsti, zeros2, dc):
    @functools.partial(
        pl.kernel,
        mesh=_mesh,
        compiler_params=pltpu.CompilerParams(use_tc_tiling_on_sc=False),
        out_type=jax.ShapeDtypeStruct((NC, N_PAD, dc), jnp.float32),
        scratch_types=[
            pltpu.VMEM((KE, CH), jnp.int32),
            pltpu.VMEM((KE, CH), jnp.int32),
            [pltpu.VMEM((CH, dc), jnp.float32) for _ in range(NBUF)],
            pltpu.VMEM_SHARED((N_PAD, dc), jnp.float32),
            pltpu.VMEM_SHARED((N_PAD, dc), jnp.float32),
            [pltpu.SemaphoreType.DMA for _ in range(NBUF)],
        ],
    )
    def k(y_h, src_h, dst_h, z_h, out, src_v, dst_v, rows, acc, y_s, sems):
        cid = lax.axis_index("c")
        sid = lax.axis_index("s")
        pltpu.sync_copy(src_h.at[sid], src_v)
        pltpu.sync_copy(dst_h.at[sid], dst_v)
        r0 = sid * RPT
        pltpu.sync_copy(z_h.at[pl.ds(r0, RPT)], acc.at[pl.ds(r0, RPT)])
        pltpu.sync_copy(y_h.at[cid, pl.ds(r0, RPT)], y_s.at[pl.ds(r0, RPT)])
        plsc.subcore_barrier()

        for b in range(NBUF):
            pltpu.make_async_copy(y_s.at[src_v.at[b]], rows[b], sems[b]).start()

        def body(g, carry):
            j = NBUF * g
            for b in range(NBUF):
                pltpu.make_async_copy(
                    y_s.at[src_v.at[j + b]], rows[b], sems[b]).wait()
                pltpu.sync_copy(rows[b], acc.at[dst_v.at[j + b]], add=True)
                pltpu.make_async_copy(
                    y_s.at[src_v.at[j + b + NBUF]], rows[b], sems[b]).start()
            return carry

        lax.fori_loop(0, KE // NBUF - 1, body, 0)
        for b in range(NBUF):
            j = KE - NBUF + b
            pltpu.make_async_copy(y_s.at[src_v.at[j]], rows[b], sems[b]).wait()
            pltpu.sync_copy(rows[b], acc.at[dst_v.at[j]], add=True)

        plsc.subcore_barrier()
        pltpu.sync_copy(acc.at[pl.ds(r0, RPT)], out.at[cid, pl.ds(r0, RPT)])

    return k(y2h, srci, dsti, zeros2)


def _tc_mm(x_pad, W1):
    def body(x_ref, w_ref, o_ref):
        o_ref[...] = jnp.dot(x_ref[...], w_ref[...],
                             preferred_element_type=jnp.float32)

    return pl.pallas_call(
        body,
        out_shape=jax.ShapeDtypeStruct((N_PAD, D_H), jnp.float32),
    )(x_pad, W1)


def _tc_scale(deg2, xw):
    def body(deg_ref, xw_ref, d_ref, y_ref):
        deg = deg_ref[0] + deg_ref[1] + 1.0
        dcol = lax.rsqrt(deg)
        d_ref[...] = dcol
        y = xw_ref[...] * dcol
        y_ref[0] = y[:, :HH]
        y_ref[1] = y[:, HH:]

    return pl.pallas_call(
        body,
        out_shape=[
            jax.ShapeDtypeStruct((N_PAD, 1), jnp.float32),
            jax.ShapeDtypeStruct((NC, N_PAD, HH), jnp.float32),
        ],
    )(deg2, xw)


def _tc_mid(acc1, y1, d, b1, W2):
    def body(acc_ref, y_ref, d_ref, b_ref, w_ref, y2_ref):
        dcol = d_ref[...]
        s = jnp.concatenate(
            [acc_ref[0] + y_ref[0], acc_ref[1] + y_ref[1]], axis=-1)
        h = jnp.maximum(s * dcol + b_ref[...], 0.0)
        y2 = jnp.dot(h, w_ref[...], preferred_element_type=jnp.float32) * dcol
        y2_ref[0] = y2[:, :HO]
        y2_ref[1] = y2[:, HO:]

    return pl.pallas_call(
        body,
        out_shape=jax.ShapeDtypeStruct((NC, N_PAD, HO), jnp.float32),
    )(acc1, y1, d, b1.reshape(1, D_H), W2)


def _tc_final(acc2, y2, d, b2):
    def body(acc_ref, y_ref, d_ref, b_ref, o_ref):
        s = jnp.concatenate(
            [acc_ref[0] + y_ref[0], acc_ref[1] + y_ref[1]], axis=-1)
        o_ref[...] = jnp.maximum(s * d_ref[...] + b_ref[...], 0.0)

    return pl.pallas_call(
        body,
        out_shape=jax.ShapeDtypeStruct((N_PAD, D_OUT), jnp.float32),
    )(acc2, y2, d, b2.reshape(1, D_OUT))


def kernel(x, edge_index, W1, b1, W2, b2):
    src = edge_index[0]
    dst = edge_index[1]
    tot = NC * NS * KD * CH
    dst_deg = jnp.concatenate(
        [dst, jnp.full((tot - E,), N, jnp.int32)]).reshape(NC, NS, KD, CH)
    tot_e = NS * KE * CH
    pad_e = tot_e - E
    srci = jnp.concatenate(
        [src, jnp.zeros((pad_e,), jnp.int32)]).reshape(NS, KE, CH)
    dsti = jnp.concatenate(
        [dst, jnp.full((pad_e,), N, jnp.int32)]).reshape(NS, KE, CH)

    x_pad = jnp.pad(x, ((0, N_PAD - N), (0, 0)))
    z1 = jnp.zeros((N_PAD,), jnp.float32)
    zH = jnp.zeros((N_PAD, HH), jnp.float32)
    zO = jnp.zeros((N_PAD, HO), jnp.float32)

    deg2 = _sc_degree(dst_deg, z1)
    xw = _tc_mm(x_pad, W1)
    d, y1 = _tc_scale(deg2.reshape(NC, N_PAD, 1), xw)
    acc1 = _sc_edge_pass(y1, srci, dsti, zH, HH)
    y2 = _tc_mid(acc1, y1, d, b1, W2)
    acc2 = _sc_edge_pass(y2, srci, dsti, zO, HO)
    out = _tc_final(acc2, y2, d, b2)
    return out[:N]


# speedup vs baseline: 2.4007x; 1.1447x over previous
import functools

import jax
import jax.numpy as jnp
from jax import lax
from jax.experimental import pallas as pl
from jax.experimental.pallas import tpu as pltpu
from jax.experimental.pallas import tpu_sc as plsc

N = 10000
E = 320000
D_IN = 128
D_H = 64
D_OUT = 32

NC = 2
NS = 16
CH = 128
KD = 80
KE = 160
N_PAD = 10240
RPT = N_PAD // NS
NBUF = 8

_mesh = plsc.VectorSubcoreMesh(core_axis_name="c", subcore_axis_name="s")


def _sc_degree(dsti, zeros1):
    @functools.partial(
        pl.kernel,
        mesh=_mesh,
        compiler_params=pltpu.CompilerParams(use_tc_tiling_on_sc=False),
        out_type=jax.ShapeDtypeStruct((NC, N_PAD), jnp.float32),
        scratch_types=[
            pltpu.VMEM((KD, CH), jnp.int32),
            pltpu.VMEM((CH,), jnp.float32),
            pltpu.VMEM_SHARED((N_PAD,), jnp.float32),
        ],
    )
    def k(dst_h, z_h, out, dst_v, ones_v, dacc):
        cid = lax.axis_index("c")
        sid = lax.axis_index("s")
        pltpu.sync_copy(dst_h.at[cid, sid], dst_v)
        for t in range(CH // 16):
            ones_v[pl.ds(16 * t, 16)] = jnp.full((16,), 1.0, jnp.float32)
        r0 = sid * RPT
        pltpu.sync_copy(z_h.at[pl.ds(r0, RPT)], dacc.at[pl.ds(r0, RPT)])
        plsc.subcore_barrier()

        def body(j, carry):
            pltpu.sync_copy(ones_v, dacc.at[dst_v.at[j]], add=True)
            return carry

        lax.fori_loop(0, KD, body, 0)
        plsc.subcore_barrier()
        pltpu.sync_copy(dacc.at[pl.ds(r0, RPT)], out.at[cid, pl.ds(r0, RPT)])

    return k(dsti, zeros1)


def _sc_edge_pass(y, srci, dsti, zeros2, dt):
    dc = dt // NC

    @functools.partial(
        pl.kernel,
        mesh=_mesh,
        compiler_params=pltpu.CompilerParams(use_tc_tiling_on_sc=False),
        out_type=jax.ShapeDtypeStruct((N_PAD, dt), jnp.float32),
        scratch_types=[
            pltpu.VMEM((KE, CH), jnp.int32),
            pltpu.VMEM((KE, CH), jnp.int32),
            [pltpu.VMEM((CH, dc), jnp.float32) for _ in range(NBUF)],
            pltpu.VMEM_SHARED((N_PAD, dc), jnp.float32),
            pltpu.VMEM_SHARED((N_PAD, dc), jnp.float32),
            [pltpu.SemaphoreType.DMA for _ in range(NBUF)],
        ],
    )
    def k(y_h, src_h, dst_h, z_h, out, src_v, dst_v, rows, acc, y_s, sems):
        cid = lax.axis_index("c")
        sid = lax.axis_index("s")
        c0 = cid * dc
        pltpu.sync_copy(src_h.at[sid], src_v)
        pltpu.sync_copy(dst_h.at[sid], dst_v)
        r0 = sid * RPT
        pltpu.sync_copy(z_h.at[pl.ds(r0, RPT)], acc.at[pl.ds(r0, RPT)])
        pltpu.sync_copy(y_h.at[pl.ds(r0, RPT), pl.ds(c0, dc)],
                        y_s.at[pl.ds(r0, RPT)])
        plsc.subcore_barrier()

        for b in range(NBUF):
            pltpu.make_async_copy(y_s.at[src_v.at[b]], rows[b], sems[b]).start()

        def body(g, carry):
            j = NBUF * g
            for b in range(NBUF):
                pltpu.make_async_copy(
                    y_s.at[src_v.at[j + b]], rows[b], sems[b]).wait()
                pltpu.sync_copy(rows[b], acc.at[dst_v.at[j + b]], add=True)
                pltpu.make_async_copy(
                    y_s.at[src_v.at[j + b + NBUF]], rows[b], sems[b]).start()
            return carry

        lax.fori_loop(0, KE // NBUF - 1, body, 0)
        for b in range(NBUF):
            j = KE - NBUF + b
            pltpu.make_async_copy(y_s.at[src_v.at[j]], rows[b], sems[b]).wait()
            pltpu.sync_copy(rows[b], acc.at[dst_v.at[j]], add=True)

        plsc.subcore_barrier()
        pltpu.sync_copy(acc.at[pl.ds(r0, RPT)],
                        out.at[pl.ds(r0, RPT), pl.ds(c0, dc)])

    return k(y, srci, dsti, zeros2)


def _tc_mm(x_pad, W1):
    def body(x_ref, w_ref, o_ref):
        o_ref[...] = jnp.dot(x_ref[...], w_ref[...],
                             preferred_element_type=jnp.float32)

    return pl.pallas_call(
        body,
        out_shape=jax.ShapeDtypeStruct((N_PAD, D_H), jnp.float32),
    )(x_pad, W1)


def _tc_scale(deg2, xw):
    def body(deg_ref, xw_ref, d_ref, y_ref):
        deg = deg_ref[0] + deg_ref[1] + 1.0
        dvec = lax.rsqrt(deg)
        d_ref[...] = dvec
        y_ref[...] = xw_ref[...] * dvec[:, None]

    return pl.pallas_call(
        body,
        out_shape=[
            jax.ShapeDtypeStruct((N_PAD,), jnp.float32),
            jax.ShapeDtypeStruct((N_PAD, D_H), jnp.float32),
        ],
    )(deg2, xw)


def _tc_mid(acc1, y1, d, b1, W2):
    def body(acc_ref, y_ref, d_ref, b_ref, w_ref, y2_ref):
        dcol = d_ref[...][:, None]
        h = jnp.maximum((acc_ref[...] + y_ref[...]) * dcol + b_ref[...], 0.0)
        y2_ref[...] = jnp.dot(
            h, w_ref[...], preferred_element_type=jnp.float32) * dcol

    return pl.pallas_call(
        body,
        out_shape=jax.ShapeDtypeStruct((N_PAD, D_OUT), jnp.float32),
    )(acc1, y1, d, b1.reshape(1, D_H), W2)


def _tc_final(acc2, y2, d, b2):
    def body(acc_ref, y_ref, d_ref, b_ref, o_ref):
        dcol = d_ref[...][:, None]
        o_ref[...] = jnp.maximum(
            (acc_ref[...] + y_ref[...]) * dcol + b_ref[...], 0.0)

    return pl.pallas_call(
        body,
        out_shape=jax.ShapeDtypeStruct((N_PAD, D_OUT), jnp.float32),
    )(acc2, y2, d, b2.reshape(1, D_OUT))


def kernel(x, edge_index, W1, b1, W2, b2):
    src = edge_index[0]
    dst = edge_index[1]
    tot = NC * NS * KD * CH
    dst_deg = jnp.concatenate(
        [dst, jnp.full((tot - E,), N, jnp.int32)]).reshape(NC, NS, KD, CH)
    tot_e = NS * KE * CH
    pad_e = tot_e - E
    srci = jnp.concatenate(
        [src, jnp.zeros((pad_e,), jnp.int32)]).reshape(NS, KE, CH)
    dsti = jnp.concatenate(
        [dst, jnp.full((pad_e,), N, jnp.int32)]).reshape(NS, KE, CH)

    x_pad = jnp.pad(x, ((0, N_PAD - N), (0, 0)))
    z1 = jnp.zeros((N_PAD,), jnp.float32)
    zH = jnp.zeros((N_PAD, D_H // NC), jnp.float32)
    zO = jnp.zeros((N_PAD, D_OUT // NC), jnp.float32)

    deg2 = _sc_degree(dst_deg, z1)
    xw = _tc_mm(x_pad, W1)
    d, y1 = _tc_scale(deg2, xw)
    s1 = _sc_edge_pass(y1, srci, dsti, zH, D_H)
    y2 = _tc_mid(s1, y1, d, b1, W2)
    s2 = _sc_edge_pass(y2, srci, dsti, zO, D_OUT)
    out = _tc_final(s2, y2, d, b2)
    return out[:N]
